# baseline (speedup 1.0000x reference)
import functools

import jax
import jax.numpy as jnp
from jax import lax
from jax.experimental import pallas as pl
from jax.experimental.pallas import tpu as pltpu
from jax.experimental.pallas import tpu_sc as plsc

B = 16384
D = 32

_info = plsc.get_sparse_core_info()
_NC = _info.num_cores
_NS = _info.num_subcores
_NW = _NC * _NS
_BPW = B // _NW


def _gather_body(u_idx, i_idx, mfu, mfi, mlu, mli, user_out, item_out,
                 uidx_v, iidx_v, r0, r1, r2, r3, s0, s1, s2, s3):
    wid = lax.axis_index("s") * _NC + lax.axis_index("c")
    base = wid * _BPW
    pltpu.sync_copy(u_idx.at[pl.ds(base, _BPW)], uidx_v)
    pltpu.sync_copy(i_idx.at[pl.ds(base, _BPW)], iidx_v)
    c0 = pltpu.async_copy(mfu.at[uidx_v], r0, s0)
    c1 = pltpu.async_copy(mfi.at[iidx_v], r1, s1)
    c2 = pltpu.async_copy(mlu.at[uidx_v], r2, s2)
    c3 = pltpu.async_copy(mli.at[iidx_v], r3, s3)
    c0.wait()
    pltpu.sync_copy(r0, user_out.at[pl.ds(base, _BPW)])
    c1.wait()
    pltpu.sync_copy(r1, item_out.at[pl.ds(base, _BPW)])
    c2.wait()
    pltpu.sync_copy(r2, user_out.at[pl.ds(B + base, _BPW)])
    c3.wait()
    pltpu.sync_copy(r3, item_out.at[pl.ds(B + base, _BPW)])


_gather = pl.kernel(
    _gather_body,
    out_type=(
        jax.ShapeDtypeStruct((2 * B, D), jnp.float32),
        jax.ShapeDtypeStruct((2 * B, D), jnp.float32),
    ),
    mesh=plsc.VectorSubcoreMesh(core_axis_name="c", subcore_axis_name="s"),
    compiler_params=pltpu.CompilerParams(use_tc_tiling_on_sc=False),
    scratch_types=[
        pltpu.VMEM((_BPW,), jnp.int32),
        pltpu.VMEM((_BPW,), jnp.int32),
        pltpu.VMEM((_BPW, D), jnp.float32),
        pltpu.VMEM((_BPW, D), jnp.float32),
        pltpu.VMEM((_BPW, D), jnp.float32),
        pltpu.VMEM((_BPW, D), jnp.float32),
        pltpu.SemaphoreType.DMA,
        pltpu.SemaphoreType.DMA,
        pltpu.SemaphoreType.DMA,
        pltpu.SemaphoreType.DMA,
    ],
)


def _mlp_body(umf, imf, umlp, imlp, W1, b1, W2, b2, Wp, bp, out):
    mf = umf[...] * imf[...]
    mlp = jnp.concatenate([umlp[...], imlp[...]], axis=1)
    h = lax.dot_general(mlp, W1[...], (((1,), (1,)), ((), ())),
                        preferred_element_type=jnp.float32) + b1[...]
    h = jnp.maximum(h, 0.0)
    h = lax.dot_general(h, W2[...], (((1,), (1,)), ((), ())),
                        preferred_element_type=jnp.float32) + b2[...]
    h = jnp.maximum(h, 0.0)
    pv = jnp.concatenate([mf, h], axis=1)
    logit = jnp.sum(pv * Wp[...], axis=1) + bp[0, 0]
    out[...] = jax.nn.sigmoid(logit)


_BLK = 2048


def _mlp(user_repr, item_repr, W1, b1, W2, b2, Wp, bp):
    nb = B // _BLK
    half = B // _BLK
    return pl.pallas_call(
        _mlp_body,
        grid=(nb,),
        in_specs=[
            pl.BlockSpec((_BLK, D), lambda i: (i, 0)),
            pl.BlockSpec((_BLK, D), lambda i: (i, 0)),
            pl.BlockSpec((_BLK, D), lambda i: (i + half, 0)),
            pl.BlockSpec((_BLK, D), lambda i: (i + half, 0)),
            pl.BlockSpec((64, 64), lambda i: (0, 0)),
            pl.BlockSpec((1, 64), lambda i: (0, 0)),
            pl.BlockSpec((32, 64), lambda i: (0, 0)),
            pl.BlockSpec((1, 32), lambda i: (0, 0)),
            pl.BlockSpec((1, 64), lambda i: (0, 0)),
            pl.BlockSpec((1, 1), lambda i: (0, 0)),
        ],
        out_specs=pl.BlockSpec((_BLK,), lambda i: (i,)),
        out_shape=jax.ShapeDtypeStruct((B,), jnp.float32),
    )(user_repr, item_repr, user_repr, item_repr,
      W1, b1.reshape(1, 64), W2, b2.reshape(1, 32), Wp, bp.reshape(1, 1))


def kernel(user_indices, item_indices, mf_user_emb, mf_item_emb,
           mlp_user_emb, mlp_item_emb, W1, b1, W2, b2, Wp, bp):
    user_repr, item_repr = _gather(
        user_indices.astype(jnp.int32), item_indices.astype(jnp.int32),
        mf_user_emb, mf_item_emb, mlp_user_emb, mlp_item_emb)
    prediction = _mlp(user_repr, item_repr, W1, b1, W2, b2, Wp, bp)
    return (prediction, user_repr, item_repr)
